# baseline (speedup 1.0000x reference)
import functools

import jax
import jax.numpy as jnp
from jax import lax
from jax.experimental import pallas as pl
from jax.experimental.pallas import tpu as pltpu
from jax.experimental.pallas import tpu_sc as plsc

NUM_EMB = 1_000_000
IDIM = 31
KDIM = 32
EDIM = 128
NQ = 4
QS = 249984
_PBN = 8064
_TAIL = NUM_EMB - NQ * QS
_NZ = 64
PROWS = QS + _TAIL + _NZ

_NC = 2
_NS = 16
_NW = _NC * _NS

_CHUNK = 128


def _tc_pack():
    nreg = QS // _PBN

    def pk(w_hbm, tail_ref, o_ref, scr, sem):
        c = pl.program_id(0)

        def dma(step, slot, q):
            return pltpu.make_async_copy(
                w_hbm.at[:, pl.ds(q * QS + step * _PBN, _PBN)],
                scr.at[slot, q],
                sem.at[slot],
            )

        @pl.when(c == 0)
        def _():
            for q in range(NQ):
                dma(0, 0, q).start()

        @pl.when(c + 1 < nreg)
        def _():
            for q in range(NQ):
                dma(c + 1, (c + 1) % 2, q).start()

        @pl.when(c < nreg)
        def _():
            slot = c % 2
            for q in range(NQ):
                dma(c, slot, q).wait()
            row = lax.broadcasted_iota(jnp.int32, (IDIM, KDIM), 0)
            col = lax.broadcasted_iota(jnp.int32, (IDIM, KDIM), 1)
            eye = (row == col).astype(jnp.float32)
            for q in range(NQ):
                o_ref[:, q * KDIM:(q + 1) * KDIM] = lax.dot_general(
                    scr[slot, q], eye, (((0,), (0,)), ((), ())),
                    preferred_element_type=jnp.float32)

        @pl.when(c == nreg)
        def _():
            o_ref[...] = jnp.pad(
                tail_ref[...],
                ((0, _PBN - _TAIL), (0, (NQ - 1) * KDIM)))

    return pl.pallas_call(
        pk,
        grid=(nreg + 1,),
        in_specs=[
            pl.BlockSpec(memory_space=pl.ANY),
            pl.BlockSpec((_TAIL, KDIM), lambda c: (0, 0)),
        ],
        out_specs=pl.BlockSpec((_PBN, NQ * KDIM), lambda c: (c, 0)),
        out_shape=jax.ShapeDtypeStruct((PROWS, NQ * KDIM), jnp.float32),
        scratch_shapes=[
            pltpu.VMEM((2, NQ, IDIM, _PBN), jnp.float32),
            pltpu.SemaphoreType.DMA((2,)),
        ],
    )


def _sc_gather(B: int):
    b_per_w = B // _NW
    n_chunks = b_per_w // _CHUNK
    mesh = plsc.VectorSubcoreMesh(core_axis_name="c", subcore_axis_name="s")

    @functools.partial(
        pl.kernel,
        mesh=mesh,
        out_type=jax.ShapeDtypeStruct((B, KDIM), jnp.float32),
        scratch_types=[
            pltpu.VMEM((b_per_w,), jnp.int32),
            [pltpu.VMEM((_CHUNK, KDIM), jnp.float32) for _ in range(2)],
            pltpu.SemaphoreType.DMA,
            pltpu.SemaphoreType.DMA,
        ],
        compiler_params=pltpu.CompilerParams(use_tc_tiling_on_sc=False),
    )
    def k(idx_hbm, table_hbm, out_hbm, idx_v, gb, sg, so):
        wid = lax.axis_index("s") * _NC + lax.axis_index("c")
        base = wid * b_per_w
        pltpu.sync_copy(idx_hbm.at[pl.ds(base, b_per_w)], idx_v)

        def out_copy(par, j):
            return pltpu.make_async_copy(
                gb[par],
                out_hbm.at[pl.ds(base + j * _CHUNK, _CHUNK), :],
                so,
            )

        def gather(par, j):
            return pltpu.async_copy(
                table_hbm.at[idx_v.at[pl.ds(j * _CHUNK, _CHUNK)]],
                gb[par],
                sg,
            )

        def pair_body(jj, carry):
            for par in range(2):
                j = jj * 2 + par

                @pl.when(jj >= 1)
                def _():
                    out_copy(par, j).wait()

                gather(par, j).wait()
                out_copy(par, j).start()
            return carry

        lax.fori_loop(0, n_chunks // 2, pair_body, 0)
        for par in range(2):
            out_copy(par, 0).wait()

    return k


def _tc_matmul(B: int, block_n: int):
    rows = B // NQ

    def mm(g_ref, w_ref, o_ref):
        o_ref[...] = jnp.dot(g_ref[...], w_ref[...],
                             preferred_element_type=jnp.float32)

    return pl.pallas_call(
        mm,
        grid=(rows // block_n,),
        in_specs=[
            pl.BlockSpec((block_n, NQ * KDIM), lambda i: (i, 0)),
            pl.BlockSpec((NQ * KDIM, NQ * EDIM), lambda i: (0, 0)),
        ],
        out_specs=pl.BlockSpec((block_n, NQ * EDIM), lambda i: (i, 0)),
        out_shape=jax.ShapeDtypeStruct((rows, NQ * EDIM), jnp.float32),
    )


def kernel(x, w1, w2):
    Bt, F = x.shape
    B = Bt * F
    idx = x.T.reshape(B).astype(jnp.int32)
    tail = idx >= NQ * QS
    ridx = jnp.where(tail, idx - NQ * QS + QS, idx % QS)
    qsel = jnp.where(tail, 0, idx // QS)
    midx = ridx * NQ + qsel
    tailp = jnp.pad(w1[NQ * QS:], ((0, 0), (0, KDIM - IDIM)))
    table = _tc_pack()(w1.T, tailp)
    tflat = table.reshape(PROWS * NQ, KDIM)
    w2p = jnp.pad(w2, ((0, KDIM - IDIM), (0, 0)))
    wbd = (jnp.eye(NQ, dtype=jnp.float32)[:, None, :, None]
           * w2p[None, :, None, :]).reshape(NQ * KDIM, NQ * EDIM)
    emb = _sc_gather(B)(midx, tflat)
    out = _tc_matmul(B, 512)(emb.reshape(B // NQ, NQ * KDIM), wbd)
    return out.reshape(B, EDIM).reshape(F, Bt, EDIM).transpose(1, 0, 2)


# speedup vs baseline: 1.0170x; 1.0170x over previous
import functools

import jax
import jax.numpy as jnp
from jax import lax
from jax.experimental import pallas as pl
from jax.experimental.pallas import tpu as pltpu
from jax.experimental.pallas import tpu_sc as plsc

NUM_EMB = 1_000_000
IDIM = 31
KDIM = 32
EDIM = 128
NQ = 4
QS = 249984
_PBN = 8064
_TAIL = NUM_EMB - NQ * QS
_NZ = 64
PROWS = QS + _TAIL + _NZ

_NC = 2
_NS = 16
_NW = _NC * _NS

_CHUNK = 128


def _tc_pack():
    nreg = QS // _PBN

    def pk(w_hbm, tail_ref, o_ref, scr, sem):
        c = pl.program_id(0)

        def dma(step, slot, q):
            return pltpu.make_async_copy(
                w_hbm.at[:, pl.ds(q * QS + step * _PBN, _PBN)],
                scr.at[slot, q],
                sem.at[slot],
            )

        @pl.when(c == 0)
        def _():
            for q in range(NQ):
                dma(0, 0, q).start()

        @pl.when(c + 1 < nreg)
        def _():
            for q in range(NQ):
                dma(c + 1, (c + 1) % 2, q).start()

        @pl.when(c < nreg)
        def _():
            slot = c % 2
            for q in range(NQ):
                dma(c, slot, q).wait()
            row = lax.broadcasted_iota(jnp.int32, (IDIM, KDIM), 0)
            col = lax.broadcasted_iota(jnp.int32, (IDIM, KDIM), 1)
            eye = (row == col).astype(jnp.float32)
            for q in range(NQ):
                o_ref[:, q * KDIM:(q + 1) * KDIM] = lax.dot_general(
                    scr[slot, q], eye, (((0,), (0,)), ((), ())),
                    preferred_element_type=jnp.float32)

        @pl.when(c == nreg)
        def _():
            o_ref[...] = jnp.pad(
                tail_ref[...],
                ((0, _PBN - _TAIL), (0, (NQ - 1) * KDIM)))

    return pl.pallas_call(
        pk,
        grid=(nreg + 1,),
        in_specs=[
            pl.BlockSpec(memory_space=pl.ANY),
            pl.BlockSpec((_TAIL, KDIM), lambda c: (0, 0)),
        ],
        out_specs=pl.BlockSpec((_PBN, NQ * KDIM), lambda c: (c, 0)),
        out_shape=jax.ShapeDtypeStruct((PROWS, NQ * KDIM), jnp.float32),
        scratch_shapes=[
            pltpu.VMEM((2, NQ, IDIM, _PBN), jnp.float32),
            pltpu.SemaphoreType.DMA((2,)),
        ],
    )


def _sc_gather(B: int):
    b_per_w = B // _NW
    n_chunks = b_per_w // _CHUNK
    mesh = plsc.VectorSubcoreMesh(core_axis_name="c", subcore_axis_name="s")

    @functools.partial(
        pl.kernel,
        mesh=mesh,
        out_type=jax.ShapeDtypeStruct((B, KDIM), jnp.float32),
        scratch_types=[
            pltpu.VMEM((b_per_w,), jnp.int32),
            [pltpu.VMEM((_CHUNK, KDIM), jnp.float32) for _ in range(2)],
            pltpu.SemaphoreType.DMA,
            pltpu.SemaphoreType.DMA,
        ],
        compiler_params=pltpu.CompilerParams(use_tc_tiling_on_sc=False),
    )
    def k(idx_hbm, table_hbm, out_hbm, idx_v, gb, sg, so):
        wid = lax.axis_index("s") * _NC + lax.axis_index("c")
        base = wid * b_per_w
        pltpu.sync_copy(idx_hbm.at[pl.ds(base, b_per_w)], idx_v)

        def out_copy(par, j):
            return pltpu.make_async_copy(
                gb[par],
                out_hbm.at[pl.ds(base + j * _CHUNK, _CHUNK), :],
                so,
            )

        def gather(par, j):
            return pltpu.async_copy(
                table_hbm.at[idx_v.at[pl.ds(j * _CHUNK, _CHUNK)]],
                gb[par],
                sg,
            )

        def pair_body(jj, carry):
            for par in range(2):
                j = jj * 2 + par

                @pl.when(jj >= 1)
                def _():
                    out_copy(par, j).wait()

                gather(par, j).wait()
                out_copy(par, j).start()
            return carry

        lax.fori_loop(0, n_chunks // 2, pair_body, 0)
        for par in range(2):
            out_copy(par, 0).wait()

    return k


def _tc_matmul(B: int, block_n: int):
    def mm(g_ref, w_ref, o_ref):
        o_ref[...] = jnp.dot(g_ref[...], w_ref[...],
                             preferred_element_type=jnp.float32)

    return pl.pallas_call(
        mm,
        grid=(B // block_n,),
        in_specs=[
            pl.BlockSpec((block_n, KDIM), lambda i: (i, 0)),
            pl.BlockSpec((KDIM, EDIM), lambda i: (0, 0)),
        ],
        out_specs=pl.BlockSpec((block_n, EDIM), lambda i: (i, 0)),
        out_shape=jax.ShapeDtypeStruct((B, EDIM), jnp.float32),
    )


def kernel(x, w1, w2):
    Bt, F = x.shape
    B = Bt * F
    idx = x.T.reshape(B).astype(jnp.int32)
    tail = idx >= NQ * QS
    ridx = jnp.where(tail, idx - NQ * QS + QS, idx % QS)
    qsel = jnp.where(tail, 0, idx // QS)
    midx = ridx * NQ + qsel
    tailp = jnp.pad(w1[NQ * QS:], ((0, 0), (0, KDIM - IDIM)))
    table = _tc_pack()(w1.T, tailp)
    tflat = table.reshape(PROWS * NQ, KDIM)
    w2p = jnp.pad(w2, ((0, KDIM - IDIM), (0, 0)))
    emb = _sc_gather(B)(midx, tflat)
    out = _tc_matmul(B, 2048)(emb, w2p)
    return out.reshape(F, Bt, EDIM).transpose(1, 0, 2)
